# baseline (speedup 1.0000x reference)
import functools

import jax
import jax.numpy as jnp
from jax import lax
from jax.experimental import pallas as pl
from jax.experimental.pallas import tpu as pltpu
from jax.experimental.pallas import tpu_sc as plsc

B = 16384
D = 32
EPS = 1e-5
L = 16
NS = 16
ROWS = B // NS
CHUNK = 128
NCHUNK = ROWS // CHUNK
U = 8


def _rsqrt(v):
    i = lax.bitcast_convert_type(v, jnp.int32)
    i = jnp.int32(0x5F3759DF) - lax.shift_right_logical(i, 1)
    y = lax.bitcast_convert_type(i, jnp.float32)
    for _ in range(3):
        y = y * (1.5 - 0.5 * v * y * y)
    return y


def kernel(x, table, gamma, beta):
    mesh = plsc.VectorSubcoreMesh(
        core_axis_name="c", subcore_axis_name="s",
        num_cores=1, num_subcores=NS)

    @functools.partial(
        pl.kernel,
        out_type=jax.ShapeDtypeStruct((B, D), jnp.float32),
        mesh=mesh,
        compiler_params=pltpu.CompilerParams(use_tc_tiling_on_sc=False),
        scratch_types=[
            pltpu.VMEM((NCHUNK, CHUNK), jnp.int32),
            pltpu.VMEM((ROWS, D), jnp.float32),
            pltpu.VMEM((4, L), jnp.float32),
            pltpu.VMEM((NS, 4, L), jnp.float32),
            pltpu.VMEM((D,), jnp.float32),
            pltpu.VMEM((D,), jnp.float32),
            pltpu.VMEM_SHARED((NS, 4, L), jnp.float32),
            pltpu.SemaphoreType.DMA,
        ],
    )
    def sc_kernel(x_hbm, table_hbm, gamma_hbm, beta_hbm, out_hbm,
                  idx_v, rows_v, part_v, all_v, g_v, b_v, shared, sem):
        wid = lax.axis_index("s")
        base = wid * ROWS

        pltpu.sync_copy(x_hbm.at[wid], idx_v)
        pltpu.sync_copy(gamma_hbm, g_v)
        pltpu.sync_copy(beta_hbm, b_v)

        copies = [
            pltpu.async_copy(table_hbm.at[idx_v.at[j]],
                             rows_v.at[pl.ds(j * CHUNK, CHUNK)], sem)
            for j in range(NCHUNK)
        ]
        for c in copies:
            c.wait()

        zero = jnp.zeros((L,), jnp.float32)

        def stats_body(it, carry):
            s0, s1, q0, q1 = carry
            for u in range(U):
                i = it * U + u
                v0 = rows_v[i, pl.ds(0, L)]
                v1 = rows_v[i, pl.ds(L, L)]
                s0 = s0 + v0
                s1 = s1 + v1
                q0 = q0 + v0 * v0
                q1 = q1 + v1 * v1
            return s0, s1, q0, q1

        s0, s1, q0, q1 = lax.fori_loop(
            0, ROWS // U, stats_body, (zero, zero, zero, zero))
        part_v[0, :] = s0
        part_v[1, :] = s1
        part_v[2, :] = q0
        part_v[3, :] = q1

        pltpu.sync_copy(part_v, shared.at[wid])
        plsc.subcore_barrier()
        pltpu.sync_copy(shared, all_v)

        s0 = zero
        s1 = zero
        q0 = zero
        q1 = zero
        for j in range(NS):
            s0 = s0 + all_v[j, 0, :]
            s1 = s1 + all_v[j, 1, :]
            q0 = q0 + all_v[j, 2, :]
            q1 = q1 + all_v[j, 3, :]

        rb = jnp.float32(1.0 / B)
        m0 = s0 * rb
        m1 = s1 * rb
        inv0 = _rsqrt(q0 * rb - m0 * m0 + jnp.float32(EPS))
        inv1 = _rsqrt(q1 * rb - m1 * m1 + jnp.float32(EPS))
        a0 = g_v[pl.ds(0, L)] * inv0
        a1 = g_v[pl.ds(L, L)] * inv1
        c0 = b_v[pl.ds(0, L)] - m0 * a0
        c1 = b_v[pl.ds(L, L)] - m1 * a1

        def norm_body(it, carry):
            for u in range(U):
                i = it * U + u
                v0 = rows_v[i, pl.ds(0, L)]
                v1 = rows_v[i, pl.ds(L, L)]
                rows_v[i, pl.ds(0, L)] = jnp.maximum(v0 * a0 + c0, 0.0)
                rows_v[i, pl.ds(L, L)] = jnp.maximum(v1 * a1 + c1, 0.0)
            return carry

        lax.fori_loop(0, ROWS // U, norm_body, 0)
        pltpu.sync_copy(rows_v, out_hbm.at[pl.ds(base, ROWS)])

    xr = x.astype(jnp.int32).reshape(NS, NCHUNK, CHUNK)
    return sc_kernel(xr, table, gamma, beta)


# speedup vs baseline: 3.1070x; 3.1070x over previous
import functools

import jax
import jax.numpy as jnp
from jax import lax
from jax.experimental import pallas as pl
from jax.experimental.pallas import tpu as pltpu
from jax.experimental.pallas import tpu_sc as plsc

B = 16384
D = 32
N = 1000000
EPS = 1e-5
L = 16
NC = 2
NS = 16
DH = D // NC
ROWS = B // NS
TW = 128
NBUF = 8


def _rsqrt(v):
    i = lax.bitcast_convert_type(v, jnp.int32)
    i = jnp.int32(0x5F3759DF) - lax.shift_right_logical(i, 1)
    y = lax.bitcast_convert_type(i, jnp.float32)
    for _ in range(3):
        y = y * (1.5 - 0.5 * v * y * y)
    return y


def kernel(x, table, gamma, beta):
    mesh = plsc.VectorSubcoreMesh(
        core_axis_name="c", subcore_axis_name="s",
        num_cores=NC, num_subcores=NS)

    @functools.partial(
        pl.kernel,
        out_type=jax.ShapeDtypeStruct((D, B), jnp.float32),
        mesh=mesh,
        compiler_params=pltpu.CompilerParams(
            use_tc_tiling_on_sc=True, needs_layout_passes=False),
        scratch_types=[
            pltpu.VMEM((ROWS + L,), jnp.int32),
            pltpu.VMEM((NBUF, DH, TW), jnp.float32),
            pltpu.VMEM((DH, ROWS), jnp.float32),
            pltpu.VMEM((8, 128), jnp.float32),
            pltpu.VMEM((NS, 8, 128), jnp.float32),
            pltpu.VMEM((D,), jnp.float32),
            pltpu.VMEM((D,), jnp.float32),
            pltpu.VMEM_SHARED((NS, 8, 128), jnp.float32),
        ] + [pltpu.SemaphoreType.DMA] * NBUF,
        )
    def sc_kernel(x_hbm, tableT_hbm, gamma_hbm, beta_hbm, outT_hbm,
                  idx_v, ring, buf, part_v, all_v, g_v, b_v, shared,
                  *sems):
        cid = lax.axis_index("c")
        sid = lax.axis_index("s")
        base = pl.multiple_of(sid * ROWS, ROWS)
        dbase = pl.multiple_of(cid * DH, DH)

        pltpu.sync_copy(x_hbm.at[pl.ds(base, ROWS)], idx_v.at[pl.ds(0, ROWS)])
        pltpu.sync_copy(gamma_hbm, g_v)
        pltpu.sync_copy(beta_hbm, b_v)

        lanes = lax.iota(jnp.int32, L)

        def issue(r_scalar, u):
            tc = pl.multiple_of((r_scalar >> 7) * TW, TW)
            return pltpu.async_copy(
                tableT_hbm.at[pl.ds(dbase, DH), pl.ds(tc, TW)],
                ring.at[u], sems[u])

        first = idx_v[pl.ds(0, L)]
        for u in range(NBUF):
            issue(first[u], u)

        GRP = ROWS // L
        zero = jnp.zeros((L,), jnp.float32)

        def body(g, carry):
            s, q = carry
            va = idx_v[pl.ds(g * L, L)]
            vb = idx_v[pl.ds(g * L + L, L)]
            for u in range(L):
                j = g * L + u
                slot = u % NBUF
                pltpu.make_async_copy(
                    tableT_hbm.at[pl.ds(dbase, DH), pl.ds(0, TW)],
                    ring.at[slot], sems[slot]).wait()
                lane = va[u] & 127
                v = plsc.load_gather(
                    ring, [jnp.full((L,), slot, jnp.int32), lanes,
                           jnp.full((L,), lane, jnp.int32)])
                s = s + v
                q = q + v * v
                plsc.store_scatter(
                    buf, [lanes, jnp.full((L,), j, jnp.int32)], v)
                if u < NBUF:
                    issue(va[u + NBUF], slot)
                else:
                    @pl.when(g < GRP - 1)
                    def _():
                        issue(vb[u - NBUF], slot)
            return s, q

        s, q = lax.fori_loop(0, GRP, body, (zero, zero))

        zi = jnp.zeros((L,), jnp.int32)
        plsc.store_scatter(part_v, [zi, lanes], s)
        plsc.store_scatter(part_v, [zi + 1, lanes], q)
        pltpu.sync_copy(part_v, shared.at[sid])
        plsc.subcore_barrier()
        pltpu.sync_copy(shared, all_v)

        s = zero
        q = zero
        for w in range(NS):
            wv = jnp.full((L,), w, jnp.int32)
            s = s + plsc.load_gather(all_v, [wv, zi, lanes])
            q = q + plsc.load_gather(all_v, [wv, zi + 1, lanes])

        rb = jnp.float32(1.0 / B)
        m = s * rb
        inv = _rsqrt(q * rb - m * m + jnp.float32(EPS))
        dsel = lanes + cid * DH
        a = plsc.load_gather(g_v, [dsel]) * inv
        cc = plsc.load_gather(b_v, [dsel]) - m * a

        def norm_body(t, carry):
            for u in range(8):
                j = t * 8 + u
                jv = jnp.full((L,), j, jnp.int32)
                v = plsc.load_gather(buf, [lanes, jv])
                plsc.store_scatter(
                    buf, [lanes, jv], jnp.maximum(v * a + cc, 0.0))
            return carry

        lax.fori_loop(0, ROWS // 8, norm_body, 0)
        pltpu.sync_copy(buf, outT_hbm.at[pl.ds(dbase, DH), pl.ds(base, ROWS)])

    tT = table.T
    outT = sc_kernel(x.astype(jnp.int32), tT, gamma, beta)
    return outT.T
